# baseline (speedup 1.0000x reference)
import jax
import jax.numpy as jnp
from jax.experimental import pallas as pl
from jax.experimental.pallas import tpu as pltpu

N_ROWS = 65536
IN_DIM = 10
OUT_DIM = 150
PACK = 8
M_PACKED = N_ROWS // PACK
K_PACKED = IN_DIM * PACK
N_PACKED = OUT_DIM * PACK
BLOCK_M = 1024


def _matmul_block(x_ref, w_ref, o_ref):
    o_ref[...] = jax.lax.dot_general(
        x_ref[...],
        w_ref[...],
        dimension_numbers=(((1,), (0,)), ((), ())),
        preferred_element_type=jnp.float32,
    )


@jax.jit
def kernel(sparse_matrix, dense_matrix):
    x_packed = sparse_matrix.reshape(M_PACKED, K_PACKED) * jnp.float32(0.5)
    w_packed = jnp.kron(jnp.eye(PACK, dtype=jnp.float32), dense_matrix)
    out_packed = pl.pallas_call(
        _matmul_block,
        grid=(M_PACKED // BLOCK_M,),
        in_specs=[
            pl.BlockSpec((BLOCK_M, K_PACKED), lambda i: (i, 0)),
            pl.BlockSpec((K_PACKED, N_PACKED), lambda i: (0, 0)),
        ],
        out_specs=pl.BlockSpec((BLOCK_M, N_PACKED), lambda i: (i, 0)),
        out_shape=jax.ShapeDtypeStruct((M_PACKED, N_PACKED), jnp.float32),
        compiler_params=pltpu.CompilerParams(
            dimension_semantics=("parallel",),
        ),
    )(x_packed, w_packed)
    return out_packed.reshape(N_ROWS, OUT_DIM) * jnp.float32(2.0)


# speedup vs baseline: 1.6967x; 1.6967x over previous
import jax
import jax.numpy as jnp
from jax.experimental import pallas as pl
from jax.experimental.pallas import tpu as pltpu

N_ROWS = 65536
IN_DIM = 10
OUT_DIM = 150
PACK = 8
M_PACKED = N_ROWS // PACK
K_PACKED = IN_DIM * PACK
BLOCK_M = 1024


def _matmul_block(x_ref, w_ref, o_ref):
    for u in range(PACK):
        o_ref[:, u, :] = jax.lax.dot_general(
            x_ref[:, u * IN_DIM:(u + 1) * IN_DIM],
            w_ref[...],
            dimension_numbers=(((1,), (0,)), ((), ())),
            preferred_element_type=jnp.float32,
        )


@jax.jit
def kernel(sparse_matrix, dense_matrix):
    x_packed = sparse_matrix.reshape(M_PACKED, K_PACKED)
    out3 = pl.pallas_call(
        _matmul_block,
        grid=(M_PACKED // BLOCK_M,),
        in_specs=[
            pl.BlockSpec((BLOCK_M, K_PACKED), lambda i: (i, 0)),
            pl.BlockSpec((IN_DIM, OUT_DIM), lambda i: (0, 0)),
        ],
        out_specs=pl.BlockSpec((BLOCK_M, PACK, OUT_DIM), lambda i: (i, 0, 0)),
        out_shape=jax.ShapeDtypeStruct((M_PACKED, PACK, OUT_DIM), jnp.float32),
        compiler_params=pltpu.CompilerParams(
            dimension_semantics=("parallel",),
        ),
    )(x_packed, dense_matrix)
    return out3.reshape(N_ROWS, OUT_DIM)


# speedup vs baseline: 2.9925x; 1.7637x over previous
import jax
import jax.numpy as jnp
from jax.experimental import pallas as pl
from jax.experimental.pallas import tpu as pltpu

N_ROWS = 65536
IN_DIM = 10
OUT_DIM = 150
BLOCK_M = 8192


def _matmul_block(xt_ref, w_ref, o_ref):
    o_ref[...] = jax.lax.dot_general(
        xt_ref[...],
        w_ref[...],
        dimension_numbers=(((0,), (0,)), ((), ())),
        preferred_element_type=jnp.float32,
    )


@jax.jit
def kernel(sparse_matrix, dense_matrix):
    xt = sparse_matrix.T
    return pl.pallas_call(
        _matmul_block,
        grid=(N_ROWS // BLOCK_M,),
        in_specs=[
            pl.BlockSpec((IN_DIM, BLOCK_M), lambda i: (0, i)),
            pl.BlockSpec((IN_DIM, OUT_DIM), lambda i: (0, 0)),
        ],
        out_specs=pl.BlockSpec((BLOCK_M, OUT_DIM), lambda i: (i, 0)),
        out_shape=jax.ShapeDtypeStruct((N_ROWS, OUT_DIM), jnp.float32),
        compiler_params=pltpu.CompilerParams(
            dimension_semantics=("parallel",),
        ),
    )(xt, dense_matrix)
